# Initial kernel scaffold; baseline (speedup 1.0000x reference)
import functools

import jax
import jax.numpy as jnp
from jax import lax
from jax.experimental import pallas as pl
from jax.experimental.pallas import tpu as pltpu
from jax.experimental.pallas import tpu_sc as plsc

NC = 2
NS = 16
NW = NC * NS

CHUNK = 1024


def _make_gather(n_rows, vocab, dim):
  assert n_rows % NW == 0
  per_w = n_rows // NW
  assert per_w % CHUNK == 0
  n_chunks = per_w // CHUNK

  mesh = plsc.VectorSubcoreMesh(core_axis_name="c", subcore_axis_name="s")

  @functools.partial(
      pl.kernel,
      out_type=jax.ShapeDtypeStruct((n_rows, dim), jnp.float32),
      mesh=mesh,
      scratch_types=[
          pltpu.VMEM((per_w,), jnp.int32),
          pltpu.VMEM((CHUNK, dim), jnp.float32),
          pltpu.SemaphoreType.DMA,
      ],
  )
  def gather_kernel(idx_hbm, table_hbm, out_hbm, idx_v, rows_v, sem):
    wid = lax.axis_index("s") * NC + lax.axis_index("c")
    base = wid * per_w
    pltpu.sync_copy(idx_hbm.at[pl.ds(base, per_w)], idx_v)

    @pl.loop(0, n_chunks)
    def _chunk(i):
      off = i * CHUNK
      pltpu.async_copy(
          table_hbm.at[idx_v.at[pl.ds(off, CHUNK)]], rows_v, sem
      ).wait()
      pltpu.sync_copy(rows_v, out_hbm.at[pl.ds(base + off, CHUNK)])

  return gather_kernel


def kernel(token_ids, weight):
  batch, hist = token_ids.shape
  vocab, dim = weight.shape
  n_rows = batch * hist
  idx = token_ids.reshape(n_rows).astype(jnp.int32)
  out = _make_gather(n_rows, vocab, dim)(idx, weight)
  return out.reshape(batch, hist, dim)


# speedup vs baseline: 1.8688x; 1.8688x over previous
import functools

import jax
import jax.numpy as jnp
from jax import lax
from jax.experimental import pallas as pl
from jax.experimental.pallas import tpu as pltpu
from jax.experimental.pallas import tpu_sc as plsc

NC = 2
NS = 16
NW = NC * NS

CHUNK = 1024


def _make_gather(n_rows, vocab, dim):
  assert n_rows % NW == 0
  per_w = n_rows // NW
  assert per_w % CHUNK == 0
  n_chunks = per_w // CHUNK

  mesh = plsc.VectorSubcoreMesh(core_axis_name="c", subcore_axis_name="s")

  @functools.partial(
      pl.kernel,
      out_type=jax.ShapeDtypeStruct((n_rows, dim), jnp.float32),
      mesh=mesh,
      scratch_types=[
          pltpu.VMEM((per_w,), jnp.int32),
          pltpu.VMEM((CHUNK, dim), jnp.float32),
          pltpu.SemaphoreType.DMA,
      ],
      compiler_params=pltpu.CompilerParams(use_tc_tiling_on_sc=False),
  )
  def gather_kernel(idx_hbm, table_hbm, out_hbm, idx_v, rows_v, sem):
    wid = lax.axis_index("s") * NC + lax.axis_index("c")
    base = wid * per_w
    pltpu.sync_copy(idx_hbm.at[pl.ds(base, per_w)], idx_v)

    @pl.loop(0, n_chunks)
    def _chunk(i):
      off = i * CHUNK
      pltpu.async_copy(
          table_hbm.at[idx_v.at[pl.ds(off, CHUNK)]], rows_v, sem
      ).wait()
      pltpu.sync_copy(rows_v, out_hbm.at[pl.ds(base + off, CHUNK)])

  return gather_kernel


def kernel(token_ids, weight):
  batch, hist = token_ids.shape
  vocab, dim = weight.shape
  n_rows = batch * hist
  idx = token_ids.reshape(n_rows).astype(jnp.int32)
  out = _make_gather(n_rows, vocab, dim)(idx, weight)
  return out.reshape(batch, hist, dim)


# speedup vs baseline: 1.8748x; 1.0032x over previous
import functools

import jax
import jax.numpy as jnp
from jax import lax
from jax.experimental import pallas as pl
from jax.experimental.pallas import tpu as pltpu
from jax.experimental.pallas import tpu_sc as plsc

NC = 2
NS = 16
NW = NC * NS

CHUNK = 400
NBUF = 4


def _make_gather(n_rows, vocab, dim):
  assert n_rows % NW == 0
  per_w = n_rows // NW
  assert per_w % (CHUNK * NBUF) == 0
  n_chunks = per_w // CHUNK

  mesh = plsc.VectorSubcoreMesh(core_axis_name="c", subcore_axis_name="s")

  @functools.partial(
      pl.kernel,
      out_type=jax.ShapeDtypeStruct((n_rows, dim), jnp.float32),
      mesh=mesh,
      scratch_types=(
          [pltpu.VMEM((per_w,), jnp.int32)]
          + [pltpu.VMEM((CHUNK, dim), jnp.float32) for _ in range(NBUF)]
          + [pltpu.SemaphoreType.DMA for _ in range(2 * NBUF)]
      ),
      compiler_params=pltpu.CompilerParams(use_tc_tiling_on_sc=False),
  )
  def gather_kernel(idx_hbm, table_hbm, out_hbm, idx_v, *bufs):
    rows = bufs[:NBUF]
    gsem = bufs[NBUF:2 * NBUF]
    ssem = bufs[2 * NBUF:]
    wid = lax.axis_index("s") * NC + lax.axis_index("c")
    base = wid * per_w

    pltpu.sync_copy(idx_hbm.at[pl.ds(base, per_w)], idx_v)

    def fire_gather(j, b):
      pltpu.async_copy(
          table_hbm.at[idx_v.at[pl.ds(j * CHUNK, CHUNK)]], rows[b], gsem[b]
      )

    def wait_gather(b):
      pltpu.make_async_copy(
          table_hbm.at[idx_v.at[pl.ds(0, CHUNK)]], rows[b], gsem[b]
      ).wait()

    def fire_scatter(j, b):
      pltpu.async_copy(
          rows[b], out_hbm.at[pl.ds(base + j * CHUNK, CHUNK)], ssem[b]
      )

    def wait_scatter(b):
      pltpu.make_async_copy(
          rows[b], out_hbm.at[pl.ds(base, CHUNK)], ssem[b]
      ).wait()

    for b in range(NBUF):
      fire_gather(b, b)

    @pl.loop(0, n_chunks - NBUF, step=NBUF)
    def _chunks(i):
      for b in range(NBUF):
        j = i + b
        wait_gather(b)
        fire_scatter(j, b)
        wait_scatter(b)
        fire_gather(j + NBUF, b)

    for b in range(NBUF):
      j = n_chunks - NBUF + b
      wait_gather(b)
      fire_scatter(j, b)
      wait_scatter(b)

  return gather_kernel


def kernel(token_ids, weight):
  batch, hist = token_ids.shape
  vocab, dim = weight.shape
  n_rows = batch * hist
  idx = token_ids.reshape(n_rows).astype(jnp.int32)
  out = _make_gather(n_rows, vocab, dim)(idx, weight)
  return out.reshape(batch, hist, dim)
